# Initial kernel scaffold; baseline (speedup 1.0000x reference)
import jax
import jax.numpy as jnp
from jax.experimental import pallas as pl


def kernel(inputs, pert_embedding, gnn_kernel, mlp_w1, mlp_b1, mlp_w2, mlp_b2, adj_rows, adj_cols, adj_vals, cond_gene_matrix, cond_gene_mask):
    raise NotImplementedError("write your pallas kernel here")


# speedup vs baseline: 8.2995x; 8.2995x over previous
import jax
import jax.numpy as jnp
from jax import lax
from jax.experimental import pallas as pl
from jax.experimental.pallas import tpu as pltpu
from jax.experimental.pallas import tpu_sc as plsc

N_GENES = 50000
EMB = 64
N_EDGES = 800000
N_COND = 2048
BATCH = 4096
MAX_G = 5

NC, NS, L = 2, 16, 16
NW = NC * NS
ZSLOT = N_COND * MAX_G
ACC_ROWS = ZSLOT + L
ACC_STRIPE = ACC_ROWS // NS
CHUNK = 256
NBLK = CHUNK // L
EDGES_PAD = 819200
EPW = EDGES_PAD // NW
NCHUNK = EPW // CHUNK
CPT = N_COND // NS
PP = 16


def _sc_body(edata_hbm, mat_hbm, emb_hbm, neg1_hbm, z_hbm,
             out_hbm,
             map_v, mat_v, ebuf, colsc_v, valsc_v, slotsc_v,
             rowbuf, slots3, gbuf, pooled_v, acc, esem, gsem, ssem):
    cid = lax.axis_index("c")
    sid = lax.axis_index("s")
    wid = cid * NS + sid
    iota = lax.iota(jnp.int32, L)

    pltpu.sync_copy(neg1_hbm, map_v)
    pltpu.sync_copy(mat_hbm, mat_v)
    pltpu.sync_copy(z_hbm, acc.at[pl.ds(sid * ACC_STRIPE, ACC_STRIPE)])

    @pl.loop(0, N_COND * MAX_G // L)
    def _(i):
        g = mat_v[pl.ds(i * L, L)]
        plsc.store_scatter(map_v, [jnp.maximum(g, 0)], iota + i * L,
                           mask=g >= 0)

    plsc.subcore_barrier()

    cbase = wid * NCHUNK
    pltpu.async_copy(edata_hbm.at[cbase], ebuf.at[0], esem)

    @pl.loop(0, NCHUNK)
    def _(ch):
        par = ch % 2
        pltpu.make_async_copy(edata_hbm.at[cbase + ch], ebuf.at[par],
                              esem).wait()

        @pl.when(ch + 1 < NCHUNK)
        def _():
            pltpu.async_copy(edata_hbm.at[cbase + ch + 1],
                             ebuf.at[(ch + 1) % 2], esem)

        def compact(j, w):
            off = pl.ds(j * L, L)
            s16 = plsc.load_gather(map_v, [ebuf[par, 0, off]])
            m = s16 >= 0
            mi = m.astype(jnp.int32)
            pos = w + jnp.cumsum(mi) - 1
            plsc.store_scatter(colsc_v, [pos], ebuf[par, 1, off], mask=m)
            plsc.store_scatter(valsc_v, [pos],
                               plsc.bitcast(ebuf[par, 2, off], jnp.float32),
                               mask=m)
            plsc.store_scatter(slotsc_v, [pos // L, pos % L], s16, mask=m)
            return w + jnp.sum(mi)

        n = lax.fori_loop(0, NBLK, compact, jnp.int32(0))

        padidx = n + iota
        plsc.store_scatter(colsc_v, [padidx], jnp.zeros((L,), jnp.int32))
        plsc.store_scatter(valsc_v, [padidx], jnp.zeros((L,), jnp.float32))
        plsc.store_scatter(slotsc_v, [padidx // L, padidx % L],
                           jnp.full((L,), ZSLOT, jnp.int32))
        nblk = (n + L - 1) // L

        @pl.loop(0, NBLK)
        def _(b):
            @pl.when(b < nblk)
            def _():
                pltpu.async_copy(emb_hbm.at[colsc_v.at[pl.ds(b * L, L)]],
                                 rowbuf.at[pl.ds(b * L, L)], gsem)

        @pl.loop(0, NBLK)
        def _(b):
            @pl.when(b < nblk)
            def _():
                pltpu.make_async_copy(
                    emb_hbm.at[colsc_v.at[pl.ds(b * L, L)]],
                    rowbuf.at[pl.ds(b * L, L)], gsem).wait()
                for j in range(L):
                    r = b * L + j
                    vv = plsc.load_gather(valsc_v,
                                          [jnp.full((L,), r, jnp.int32)])
                    for q in range(EMB // L):
                        sl = pl.ds(q * L, L)
                        rowbuf[r, sl] = rowbuf[r, sl] * vv
                pltpu.async_copy(rowbuf.at[pl.ds(b * L, L)],
                                 acc.at[slotsc_v.at[b]], ssem, add=True)

        @pl.loop(0, NBLK)
        def _(b):
            @pl.when(b < nblk)
            def _():
                pltpu.make_async_copy(rowbuf.at[pl.ds(b * L, L)],
                                      acc.at[slotsc_v.at[b]], ssem).wait()

    plsc.subcore_barrier()

    @pl.loop(0, CPT // PP)
    def _(h):
        c0 = sid * CPT + h * PP
        sbase = c0 * MAX_G

        @pl.loop(0, PP * MAX_G // L)
        def _(i):
            g = mat_v[pl.ds(sbase + i * L, L)]
            s = plsc.load_gather(map_v, [jnp.maximum(g, 0)])
            slots3[pl.ds(i * L, L)] = jnp.where(g >= 0, s, ZSLOT)

        pltpu.sync_copy(acc.at[slots3], gbuf)

        @pl.loop(0, PP)
        def _(cc):
            b5 = cc * MAX_G
            for q in range(EMB // L):
                sl = pl.ds(q * L, L)
                ssum = gbuf[b5, sl]
                for j in range(1, MAX_G):
                    ssum = ssum + gbuf[b5 + j, sl]
                pooled_v[cc, sl] = ssum

        pltpu.sync_copy(pooled_v, out_hbm.at[cid, pl.ds(c0, PP)])


_sc_mesh = plsc.VectorSubcoreMesh(core_axis_name="c", subcore_axis_name="s")
_sc_params = pltpu.CompilerParams(needs_layout_passes=False,
                                  use_tc_tiling_on_sc=False)

_agg_pool = pl.kernel(
    _sc_body,
    out_type=jax.ShapeDtypeStruct((NC, N_COND, EMB), jnp.float32),
    mesh=_sc_mesh,
    compiler_params=_sc_params,
    scratch_types=[
        pltpu.VMEM((N_GENES,), jnp.int32),
        pltpu.VMEM((N_COND * MAX_G,), jnp.int32),
        pltpu.VMEM((2, 3, CHUNK), jnp.int32),
        pltpu.VMEM((CHUNK + L,), jnp.int32),
        pltpu.VMEM((CHUNK + L,), jnp.float32),
        pltpu.VMEM((NBLK + 1, L), jnp.int32),
        pltpu.VMEM((CHUNK, EMB), jnp.float32),
        pltpu.VMEM((PP * MAX_G,), jnp.int32),
        pltpu.VMEM((PP * MAX_G, EMB), jnp.float32),
        pltpu.VMEM((PP, EMB), jnp.float32),
        pltpu.VMEM_SHARED((ACC_ROWS, EMB), jnp.float32),
        pltpu.SemaphoreType.DMA,
        pltpu.SemaphoreType.DMA,
        pltpu.SemaphoreType.DMA,
    ],
)


def _tc_body(pp_ref, gnn_ref, w1_ref, b1_ref, w2_ref, b2_ref, mask_ref,
             o_ref):
    p = pp_ref[0] + pp_ref[1]
    summed = jnp.dot(p, gnn_ref[...], preferred_element_type=jnp.float32)
    h = jnp.maximum(
        jnp.dot(summed, w1_ref[...], preferred_element_type=jnp.float32)
        + b1_ref[...], 0.0)
    h = jnp.maximum(
        jnp.dot(h, w2_ref[...], preferred_element_type=jnp.float32)
        + b2_ref[...], 0.0)
    ng = jnp.sum(mask_ref[...], axis=1, keepdims=True)
    o_ref[...] = jnp.where(ng == 0.0, 0.0, jnp.where(ng == 1.0, summed, h))


_mlp = pl.pallas_call(
    _tc_body,
    out_shape=jax.ShapeDtypeStruct((N_COND, EMB), jnp.float32),
)


def _gat_body(tab_hbm, idx_hbm, out_hbm, idx_v, row_v, sem):
    wid = lax.axis_index("c") * NS + lax.axis_index("s")
    base = wid * (BATCH // NW)
    pltpu.sync_copy(idx_hbm.at[pl.ds(base, BATCH // NW)], idx_v)
    pltpu.async_copy(tab_hbm.at[idx_v], row_v, sem).wait()
    pltpu.sync_copy(row_v, out_hbm.at[pl.ds(base, BATCH // NW)])


_expand = pl.kernel(
    _gat_body,
    out_type=jax.ShapeDtypeStruct((BATCH, EMB), jnp.float32),
    mesh=_sc_mesh,
    compiler_params=_sc_params,
    scratch_types=[
        pltpu.VMEM((BATCH // NW,), jnp.int32),
        pltpu.VMEM((BATCH // NW, EMB), jnp.float32),
        pltpu.SemaphoreType.DMA,
    ],
)


def kernel(inputs, pert_embedding, gnn_kernel, mlp_w1, mlp_b1, mlp_w2, mlp_b2,
           adj_rows, adj_cols, adj_vals, cond_gene_matrix, cond_gene_mask):
    pad = EDGES_PAD - N_EDGES
    rows_p = jnp.concatenate([adj_rows, jnp.zeros((pad,), jnp.int32)])
    cols_p = jnp.concatenate([adj_cols, jnp.zeros((pad,), jnp.int32)])
    vals_p = jnp.concatenate(
        [lax.bitcast_convert_type(adj_vals, jnp.int32),
         jnp.zeros((pad,), jnp.int32)])
    edata = jnp.stack([rows_p.reshape(-1, CHUNK), cols_p.reshape(-1, CHUNK),
                       vals_p.reshape(-1, CHUNK)], axis=1)
    mat_flat = cond_gene_matrix.reshape(-1)
    neg1 = jnp.full((N_GENES,), -1, jnp.int32)
    zrows = jnp.zeros((ACC_STRIPE, EMB), jnp.float32)

    pooled_partial = _agg_pool(edata, mat_flat, pert_embedding, neg1, zrows)
    mask8 = jnp.pad(cond_gene_mask, ((0, 0), (0, 3)))
    out_cond = _mlp(pooled_partial, gnn_kernel, mlp_w1,
                    mlp_b1.reshape(1, EMB), mlp_w2, mlp_b2.reshape(1, EMB),
                    mask8)
    return _expand(out_cond, inputs.astype(jnp.int32))
